# baseline (speedup 1.0000x reference)
import functools

import jax
import jax.numpy as jnp
from jax.experimental import pallas as pl

DIM = 4096
NUM_EXPERTS = 64
BLOCK_T = 1024


def _router_body(x_ref, w_ref, topi_ref, wts_ref, pri_ref, *, block_t):
    logits = jax.lax.dot_general(
        x_ref[...], w_ref[...],
        dimension_numbers=(((1,), (0,)), ((), ())),
        preferred_element_type=jnp.float32,
    )
    m = jnp.max(logits, axis=1, keepdims=True)
    idx = jnp.argmax(logits, axis=1)
    s = jnp.sum(jnp.exp(logits - m), axis=1, keepdims=True)
    topi_ref[...] = idx[:, None].astype(jnp.int32)
    wts_ref[...] = 1.0 / s
    row0 = pl.program_id(0) * block_t
    rows = row0 + jax.lax.broadcasted_iota(jnp.int32, (block_t, 1), 0)
    pri_ref[...] = -rows.astype(jnp.float32)


@jax.jit
def kernel(x, gate_w):
    t = x.shape[0]
    grid = (t // BLOCK_T,)
    topi, wts, pri = pl.pallas_call(
        functools.partial(_router_body, block_t=BLOCK_T),
        grid=grid,
        in_specs=[
            pl.BlockSpec((BLOCK_T, DIM), lambda i: (i, 0)),
            pl.BlockSpec((DIM, NUM_EXPERTS), lambda i: (0, 0)),
        ],
        out_specs=[
            pl.BlockSpec((BLOCK_T, 1), lambda i: (i, 0)),
            pl.BlockSpec((BLOCK_T, 1), lambda i: (i, 0)),
            pl.BlockSpec((BLOCK_T, 1), lambda i: (i, 0)),
        ],
        out_shape=[
            jax.ShapeDtypeStruct((t, 1), jnp.int32),
            jax.ShapeDtypeStruct((t, 1), jnp.float32),
            jax.ShapeDtypeStruct((t, 1), jnp.float32),
        ],
    )(x, gate_w.T)
    return (topi, wts, pri.reshape(t))


# speedup vs baseline: 1.0057x; 1.0057x over previous
import functools

import jax
import jax.numpy as jnp
from jax.experimental import pallas as pl

DIM = 4096
NUM_EXPERTS = 64
BLOCK_T = 512


def _router_body(x_ref, w_ref, topi_ref, wts_ref, pri_ref, *, block_t):
    logits = jax.lax.dot_general(
        x_ref[...], w_ref[...],
        dimension_numbers=(((1,), (1,)), ((), ())),
        preferred_element_type=jnp.float32,
    )
    m = jnp.max(logits, axis=1, keepdims=True)
    idx = jnp.argmax(logits, axis=1)
    s = jnp.sum(jnp.exp(logits - m), axis=1, keepdims=True)
    topi_ref[...] = idx[:, None].astype(jnp.int32)
    wts_ref[...] = 1.0 / s
    row0 = pl.program_id(0) * block_t
    rows = row0 + jax.lax.broadcasted_iota(jnp.int32, (block_t, 1), 0)
    pri_ref[...] = -rows.astype(jnp.float32)


@jax.jit
def kernel(x, gate_w):
    t = x.shape[0]
    grid = (t // BLOCK_T,)
    topi, wts, pri = pl.pallas_call(
        functools.partial(_router_body, block_t=BLOCK_T),
        grid=grid,
        in_specs=[
            pl.BlockSpec((BLOCK_T, DIM), lambda i: (i, 0)),
            pl.BlockSpec((NUM_EXPERTS, DIM), lambda i: (0, 0)),
        ],
        out_specs=[
            pl.BlockSpec((BLOCK_T, 1), lambda i: (i, 0)),
            pl.BlockSpec((BLOCK_T, 1), lambda i: (i, 0)),
            pl.BlockSpec((BLOCK_T, 1), lambda i: (i, 0)),
        ],
        out_shape=[
            jax.ShapeDtypeStruct((t, 1), jnp.int32),
            jax.ShapeDtypeStruct((t, 1), jnp.float32),
            jax.ShapeDtypeStruct((t, 1), jnp.float32),
        ],
    )(x, gate_w)
    return (topi, wts, pri.reshape(t))


# speedup vs baseline: 1.0183x; 1.0125x over previous
import functools

import jax
import jax.numpy as jnp
from jax.experimental import pallas as pl
from jax.experimental.pallas import tpu as pltpu

DIM = 4096
NUM_EXPERTS = 64
BLOCK_T = 1024
KHALF = DIM // 2


def _router_body(x0_ref, x1_ref, w_ref, topi_ref, wts_ref, pri_ref, *, block_t):
    dn = (((1,), (1,)), ((), ()))
    logits = jax.lax.dot_general(
        x0_ref[...], w_ref[:, :KHALF], dn, preferred_element_type=jnp.float32)
    logits += jax.lax.dot_general(
        x1_ref[...], w_ref[:, KHALF:], dn, preferred_element_type=jnp.float32)
    m = jnp.max(logits, axis=1, keepdims=True)
    idx = jnp.argmax(logits, axis=1)
    s = jnp.sum(jnp.exp(logits - m), axis=1, keepdims=True)
    topi_ref[...] = idx[:, None].astype(jnp.int32)
    wts_ref[...] = 1.0 / s
    row0 = pl.program_id(0) * block_t
    rows = row0 + jax.lax.broadcasted_iota(jnp.int32, (block_t, 1), 0)
    pri_ref[...] = -rows.astype(jnp.float32)


@jax.jit
def kernel(x, gate_w):
    t = x.shape[0]
    grid = (t // BLOCK_T,)
    topi, wts, pri = pl.pallas_call(
        functools.partial(_router_body, block_t=BLOCK_T),
        grid=grid,
        in_specs=[
            pl.BlockSpec((BLOCK_T, KHALF), lambda i: (i, 0)),
            pl.BlockSpec((BLOCK_T, KHALF), lambda i: (i, 1)),
            pl.BlockSpec((NUM_EXPERTS, DIM), lambda i: (0, 0)),
        ],
        out_specs=[
            pl.BlockSpec((BLOCK_T, 1), lambda i: (i, 0)),
            pl.BlockSpec((BLOCK_T, 1), lambda i: (i, 0)),
            pl.BlockSpec((BLOCK_T, 1), lambda i: (i, 0)),
        ],
        out_shape=[
            jax.ShapeDtypeStruct((t, 1), jnp.int32),
            jax.ShapeDtypeStruct((t, 1), jnp.float32),
            jax.ShapeDtypeStruct((t, 1), jnp.float32),
        ],
        compiler_params=pltpu.CompilerParams(
            vmem_limit_bytes=128 * 1024 * 1024),
    )(x, x, gate_w)
    return (topi, wts, pri.reshape(t))


# speedup vs baseline: 1.1930x; 1.1716x over previous
import functools

import jax
import jax.numpy as jnp
from jax.experimental import pallas as pl
from jax.experimental.pallas import tpu as pltpu

DIM = 4096
NUM_EXPERTS = 64
BLOCK_T = 1024


def _router_body(x_ref, w_ref, topi_ref, wts_ref, pri_ref, *, block_t):
    logits = jax.lax.dot_general(
        x_ref[...], w_ref[...],
        dimension_numbers=(((1,), (1,)), ((), ())),
        preferred_element_type=jnp.float32,
    )
    m = jnp.max(logits, axis=1, keepdims=True)
    idx = jnp.argmax(logits, axis=1)
    s = jnp.sum(jnp.exp(logits - m), axis=1)
    topi_ref[...] = idx.astype(jnp.int32).reshape(1, 1, block_t)
    wts_ref[...] = (1.0 / s).reshape(1, 1, block_t)
    row0 = pl.program_id(0) * block_t
    rows = row0 + jax.lax.broadcasted_iota(jnp.int32, (1, 1, block_t), 2)
    pri_ref[...] = -rows.astype(jnp.float32)


@jax.jit
def kernel(x, gate_w):
    t = x.shape[0]
    nb = t // BLOCK_T
    grid = (nb,)
    topi, wts, pri = pl.pallas_call(
        functools.partial(_router_body, block_t=BLOCK_T),
        grid=grid,
        in_specs=[
            pl.BlockSpec((BLOCK_T, DIM), lambda i: (i, 0)),
            pl.BlockSpec((NUM_EXPERTS, DIM), lambda i: (0, 0)),
        ],
        out_specs=[
            pl.BlockSpec((1, 1, BLOCK_T), lambda i: (i, 0, 0)),
            pl.BlockSpec((1, 1, BLOCK_T), lambda i: (i, 0, 0)),
            pl.BlockSpec((1, 1, BLOCK_T), lambda i: (i, 0, 0)),
        ],
        out_shape=[
            jax.ShapeDtypeStruct((nb, 1, BLOCK_T), jnp.int32),
            jax.ShapeDtypeStruct((nb, 1, BLOCK_T), jnp.float32),
            jax.ShapeDtypeStruct((nb, 1, BLOCK_T), jnp.float32),
        ],
        compiler_params=pltpu.CompilerParams(
            vmem_limit_bytes=128 * 1024 * 1024),
    )(x, gate_w)
    return (topi.reshape(t, 1), wts.reshape(t, 1), pri.reshape(t))
